# Initial kernel scaffold; baseline (speedup 1.0000x reference)
import jax
import jax.numpy as jnp
from jax.experimental import pallas as pl


def kernel(inputs, targets, alpha):
    raise NotImplementedError("write your pallas kernel here")


# speedup vs baseline: 1.9590x; 1.9590x over previous
import functools

import jax
import jax.numpy as jnp
from jax import lax
from jax.experimental import pallas as pl
from jax.experimental.pallas import tpu as pltpu
from jax.experimental.pallas import tpu_sc as plsc

_NC = 2
_NS = 16
_NW = _NC * _NS
_L = 16
_CHUNK = 128


def _sc_gather(x_flat, tgt3, alpha_flat, n, c):
    b_per_w = n // _NW
    n_chunks = b_per_w // _CHUNK
    out_sd = jax.ShapeDtypeStruct((_NW, n_chunks, _CHUNK), jnp.float32)
    mesh = plsc.VectorSubcoreMesh(core_axis_name="c", subcore_axis_name="s")

    @functools.partial(
        pl.kernel,
        out_type=(out_sd, out_sd),
        mesh=mesh,
        scratch_types=[
            pltpu.VMEM((n_chunks, _CHUNK), jnp.int32),
            pltpu.VMEM((n_chunks, _CHUNK), jnp.int32),
            pltpu.VMEM((n_chunks, _CHUNK), jnp.float32),
            pltpu.VMEM((n_chunks, _CHUNK), jnp.float32),
            pltpu.SemaphoreType.DMA,
        ],
    )
    def k(x_hbm, tgt_hbm, alpha_hbm, vals_out, avals_out,
          tgt_v, idx_v, x_v, a_v, sem):
        wid = lax.axis_index("s") * _NC + lax.axis_index("c")
        base = wid * b_per_w
        pltpu.sync_copy(tgt_hbm.at[wid], tgt_v)
        dmas = []
        for j in range(n_chunks):
            dmas.append(pltpu.async_copy(alpha_hbm.at[tgt_v.at[j]], a_v.at[j], sem))
        for j in range(n_chunks):
            for l in range(_CHUNK // _L):
                t16 = tgt_v[j, pl.ds(l * _L, _L)]
                row = (base + j * _CHUNK + l * _L) + lax.iota(jnp.int32, _L)
                idx_v[j, pl.ds(l * _L, _L)] = row * c + t16
        for j in range(n_chunks):
            dmas.append(pltpu.async_copy(x_hbm.at[idx_v.at[j]], x_v.at[j], sem))
        for d in dmas:
            d.wait()
        pltpu.sync_copy(x_v, vals_out.at[wid])
        pltpu.sync_copy(a_v, avals_out.at[wid])

    return k(x_flat, tgt3, alpha_flat)


def _loss_body(v_ref, a_ref, o_ref, *, inv_n):
    p = jax.nn.sigmoid(v_ref[...])
    o_ref[...] = (-jnp.sum(a_ref[...] * jnp.log(p)) * inv_n).reshape(1, 1)


def kernel(inputs, targets, alpha):
    n, c = inputs.shape
    x_flat = inputs.reshape(-1)
    alpha_flat = alpha.reshape(-1).astype(jnp.float32)
    b_per_w = n // _NW
    n_chunks = b_per_w // _CHUNK
    tgt3 = targets.astype(jnp.int32).reshape(_NW, n_chunks, _CHUNK)

    vals, avals = _sc_gather(x_flat, tgt3, alpha_flat, n, c)

    rows = n // 128
    v2 = vals.reshape(rows, 128)
    a2 = avals.reshape(rows, 128)
    out = pl.pallas_call(
        functools.partial(_loss_body, inv_n=1.0 / n),
        out_shape=jax.ShapeDtypeStruct((1, 1), jnp.float32),
    )(v2, a2)
    return out[0, 0]


# speedup vs baseline: 2.7575x; 1.4076x over previous
import functools

import jax
import jax.numpy as jnp
from jax import lax
from jax.experimental import pallas as pl
from jax.experimental.pallas import tpu as pltpu
from jax.experimental.pallas import tpu_sc as plsc

_NC = 2
_NS = 16
_NW = _NC * _NS
_L = 16
_CHUNK = 128
_RING = 4


def _sc_gather(x, tgt3, alpha_flat, n, c):
    b_per_w = n // _NW
    n_chunks = b_per_w // _CHUNK
    n_steps = b_per_w // _L
    out_sd = jax.ShapeDtypeStruct((_NW, n_chunks, _CHUNK), jnp.float32)
    mesh = plsc.VectorSubcoreMesh(core_axis_name="c", subcore_axis_name="s")

    @functools.partial(
        pl.kernel,
        out_type=(out_sd, out_sd),
        mesh=mesh,
        compiler_params=pltpu.CompilerParams(
            use_tc_tiling_on_sc=True, needs_layout_passes=False),
        scratch_types=[
            pltpu.VMEM((n_chunks, _CHUNK), jnp.int32),
            pltpu.VMEM((n_chunks, _CHUNK), jnp.float32),
            pltpu.VMEM((n_chunks, _CHUNK), jnp.float32),
            pltpu.VMEM((_RING, _L, c), jnp.float32),
            pltpu.SemaphoreType.DMA,
            pltpu.SemaphoreType.DMA,
            pltpu.SemaphoreType.DMA,
            pltpu.SemaphoreType.DMA,
            pltpu.SemaphoreType.DMA,
        ],
    )
    def k(x_hbm, tgt_hbm, alpha_hbm, vals_out, avals_out,
          tgt_v, a_v, vals_v, slab_v, s0, s1, s2, s3, sem_a):
        sems = (s0, s1, s2, s3)
        wid = lax.axis_index("s") * _NC + lax.axis_index("c")
        base = wid * b_per_w
        pltpu.sync_copy(tgt_hbm.at[wid], tgt_v)
        a_dmas = [
            pltpu.async_copy(alpha_hbm.at[tgt_v.at[j]], a_v.at[j], sem_a)
            for j in range(n_chunks)
        ]
        dmas = [
            pltpu.async_copy(
                x_hbm.at[pl.ds(base + g * _L, _L), :], slab_v.at[g], sems[g])
            for g in range(_RING)
        ]
        row_iota = lax.iota(jnp.int32, _L)
        for s in range(n_steps):
            g = s % _RING
            dmas[g].wait()
            t16 = tgt_v[s // 8, pl.ds((s % 8) * _L, _L)]
            val16 = plsc.load_gather(slab_v.at[g], [row_iota, t16])
            vals_v[s // 8, pl.ds((s % 8) * _L, _L)] = val16
            nxt = s + _RING
            if nxt < n_steps:
                dmas[g] = pltpu.async_copy(
                    x_hbm.at[pl.ds(base + nxt * _L, _L), :], slab_v.at[g],
                    sems[g])
        for d in a_dmas:
            d.wait()
        pltpu.sync_copy(vals_v, vals_out.at[wid])
        pltpu.sync_copy(a_v, avals_out.at[wid])

    return k(x, tgt3, alpha_flat)


def _loss_body(v_ref, a_ref, o_ref, *, inv_n):
    p = jax.nn.sigmoid(v_ref[...])
    o_ref[...] = (-jnp.sum(a_ref[...] * jnp.log(p)) * inv_n).reshape(1, 1)


def kernel(inputs, targets, alpha):
    n, c = inputs.shape
    alpha_flat = alpha.reshape(-1).astype(jnp.float32)
    b_per_w = n // _NW
    n_chunks = b_per_w // _CHUNK
    tgt3 = targets.astype(jnp.int32).reshape(_NW, n_chunks, _CHUNK)

    vals, avals = _sc_gather(inputs, tgt3, alpha_flat, n, c)

    rows = n // 128
    v2 = vals.reshape(rows, 128)
    a2 = avals.reshape(rows, 128)
    out = pl.pallas_call(
        functools.partial(_loss_body, inv_n=1.0 / n),
        out_shape=jax.ShapeDtypeStruct((1, 1), jnp.float32),
    )(v2, a2)
    return out[0, 0]


# speedup vs baseline: 2.7581x; 1.0002x over previous
import functools

import jax
import jax.numpy as jnp
from jax import lax
from jax.experimental import pallas as pl
from jax.experimental.pallas import tpu as pltpu
from jax.experimental.pallas import tpu_sc as plsc

_NC = 2
_NS = 16
_NW = _NC * _NS
_L = 16
_CHUNK = 128
_RING = 4


def _sc_gather(x, targets, alpha_flat, n, c):
    b_per_w = n // _NW
    n_chunks = b_per_w // _CHUNK
    n_steps = b_per_w // _L
    out_sd = jax.ShapeDtypeStruct((n // _CHUNK, _CHUNK), jnp.float32)
    mesh = plsc.VectorSubcoreMesh(core_axis_name="c", subcore_axis_name="s")

    @functools.partial(
        pl.kernel,
        out_type=(out_sd, out_sd),
        mesh=mesh,
        compiler_params=pltpu.CompilerParams(
            use_tc_tiling_on_sc=True, needs_layout_passes=False),
        scratch_types=[
            pltpu.VMEM((b_per_w,), jnp.int32),
            pltpu.VMEM((n_chunks, _CHUNK), jnp.float32),
            pltpu.VMEM((n_chunks, _CHUNK), jnp.float32),
            pltpu.VMEM((_RING, _L, c), jnp.float32),
            pltpu.SemaphoreType.DMA,
            pltpu.SemaphoreType.DMA,
            pltpu.SemaphoreType.DMA,
            pltpu.SemaphoreType.DMA,
            pltpu.SemaphoreType.DMA,
        ],
    )
    def k(x_hbm, tgt_hbm, alpha_hbm, vals_out, avals_out,
          tgt_v, a_v, vals_v, slab_v, s0, s1, s2, s3, sem_a):
        sems = (s0, s1, s2, s3)
        wid = lax.axis_index("s") * _NC + lax.axis_index("c")
        base = wid * b_per_w
        pltpu.sync_copy(tgt_hbm.at[pl.ds(base, b_per_w)], tgt_v)
        a_dmas = [
            pltpu.async_copy(
                alpha_hbm.at[tgt_v.at[pl.ds(j * _CHUNK, _CHUNK)]],
                a_v.at[j], sem_a)
            for j in range(n_chunks)
        ]
        dmas = [
            pltpu.async_copy(
                x_hbm.at[pl.ds(base + g * _L, _L), :], slab_v.at[g], sems[g])
            for g in range(_RING)
        ]
        row_iota = lax.iota(jnp.int32, _L)
        for s in range(n_steps):
            g = s % _RING
            dmas[g].wait()
            t16 = tgt_v[pl.ds(s * _L, _L)]
            val16 = plsc.load_gather(slab_v.at[g], [row_iota, t16])
            vals_v[s // 8, pl.ds((s % 8) * _L, _L)] = val16
            nxt = s + _RING
            if nxt < n_steps:
                dmas[g] = pltpu.async_copy(
                    x_hbm.at[pl.ds(base + nxt * _L, _L), :], slab_v.at[g],
                    sems[g])
        for d in a_dmas:
            d.wait()
        pltpu.sync_copy(vals_v, vals_out.at[pl.ds(wid * n_chunks, n_chunks), :])
        pltpu.sync_copy(a_v, avals_out.at[pl.ds(wid * n_chunks, n_chunks), :])

    return k(x, targets, alpha_flat)


def _loss_body(v_ref, a_ref, o_ref, *, inv_n):
    p = jax.nn.sigmoid(v_ref[...])
    o_ref[...] = (-jnp.sum(a_ref[...] * jnp.log(p)) * inv_n).reshape(1, 1)


def kernel(inputs, targets, alpha):
    n, c = inputs.shape
    alpha_flat = alpha.reshape(-1)
    tgt = targets.astype(jnp.int32)

    vals, avals = _sc_gather(inputs, tgt, alpha_flat, n, c)

    out = pl.pallas_call(
        functools.partial(_loss_body, inv_n=1.0 / n),
        out_shape=jax.ShapeDtypeStruct((1, 1), jnp.float32),
    )(vals, avals)
    return out[0, 0]
